# Initial kernel scaffold; baseline (speedup 1.0000x reference)
import functools

import jax
import jax.numpy as jnp
from jax import lax
from jax.experimental import pallas as pl
from jax.experimental.pallas import tpu as pltpu
from jax.experimental.pallas import tpu_sc as plsc

N = 10000
E = 160000
FEAT = 3244
MB = 1000
FB = 128
CH = 80
NCHUNK = E // CH
SC_TILES = 16
ROWS_PER_TILE = NCHUNK // SC_TILES

DIMS_K = [FEAT, 2000, 1000, 500, 100, 10]
DIMS_F = [2000, 1000, 500, 100, 10, 1]
KPAD = [3328, 2048, 1024, 512, 128, 128]
FPAD = [2048, 1024, 512, 128, 128, 128]
NKB = [kp // FB for kp in KPAD]
NFB = [fp // FB for fp in FPAD]

_SC_MESH = plsc.VectorSubcoreMesh(
    core_axis_name="c", subcore_axis_name="s", num_cores=2, num_subcores=16)


def _degree_kernel(dst2d, ones_n):
    @functools.partial(
        pl.kernel,
        out_type=jax.ShapeDtypeStruct((N,), jnp.float32),
        mesh=_SC_MESH,
        scratch_types=[
            pltpu.VMEM_SHARED((N,), jnp.float32),
            pltpu.VMEM((ROWS_PER_TILE, CH), jnp.int32),
            pltpu.VMEM((CH,), jnp.float32),
        ],
    )
    def deg_kernel(dst_hbm, ones_hbm, deg_hbm, deg_sp, dst_v, ones_v):
        c = lax.axis_index("c")
        s = lax.axis_index("s")

        @pl.when(c == 0)
        def _():
            pltpu.sync_copy(dst_hbm.at[pl.ds(s * ROWS_PER_TILE, ROWS_PER_TILE), :],
                            dst_v)
            pltpu.sync_copy(ones_hbm.at[pl.ds(0, CH)], ones_v)

            @pl.when(s == 0)
            def _():
                pltpu.sync_copy(ones_hbm, deg_sp)

            plsc.subcore_barrier()

            def body(j, carry):
                pltpu.sync_copy(ones_v, deg_sp.at[dst_v.at[j]], add=True)
                return carry

            lax.fori_loop(0, ROWS_PER_TILE, body, 0)
            plsc.subcore_barrier()

            @pl.when(s == 0)
            def _():
                pltpu.sync_copy(deg_sp, deg_hbm)

    return deg_kernel(dst2d, ones_n)


def _make_scatter(n_fb):
    @functools.partial(
        pl.kernel,
        out_type=jax.ShapeDtypeStruct((n_fb, N, FB), jnp.float32),
        mesh=_SC_MESH,
        scratch_types=[
            pltpu.VMEM_SHARED((N, FB), jnp.float32),
            pltpu.VMEM((ROWS_PER_TILE, CH), jnp.int32),
            pltpu.VMEM((ROWS_PER_TILE, CH), jnp.int32),
            pltpu.VMEM((CH, FB), jnp.float32),
            pltpu.SemaphoreType.DMA,
        ],
    )
    def scat_kernel(hs_hbm, src_hbm, dst_hbm, acc_hbm,
                    acc_sp, src_v, dst_v, rows_v, sem):
        c = lax.axis_index("c")
        s = lax.axis_index("s")
        row0 = s * ROWS_PER_TILE
        pltpu.sync_copy(src_hbm.at[pl.ds(row0, ROWS_PER_TILE), :], src_v)
        pltpu.sync_copy(dst_hbm.at[pl.ds(row0, ROWS_PER_TILE), :], dst_v)
        for fb in range(n_fb):
            owner = fb % 2

            @pl.when(c == owner)
            def _(fb=fb):
                @pl.when(s == 0)
                def _():
                    pltpu.sync_copy(hs_hbm.at[fb], acc_sp)

                plsc.subcore_barrier()

                def body(j, carry):
                    pltpu.async_copy(
                        hs_hbm.at[fb].at[src_v.at[j]], rows_v, sem).wait()
                    pltpu.sync_copy(rows_v, acc_sp.at[dst_v.at[j]], add=True)
                    return carry

                lax.fori_loop(0, ROWS_PER_TILE, body, 0)
                plsc.subcore_barrier()

                @pl.when(s == 0)
                def _():
                    pltpu.sync_copy(acc_sp, acc_hbm.at[fb])

                plsc.subcore_barrier()

    return scat_kernel


_SCATTER = {n_fb: _make_scatter(n_fb) for n_fb in sorted(set(NFB))}


def _norm_dinv(x, deg2d):
    def body(x_ref, deg_ref, nm2_ref, dinv_ref, acc_ref):
        i = pl.program_id(0)

        @pl.when(i == 0)
        def _():
            acc_ref[0, 0] = 0.0

        xb = x_ref[...]
        acc_ref[0, 0] += jnp.sum(xb * xb)
        dinv_ref[...] = lax.rsqrt(deg_ref[...])

        @pl.when(i == pl.num_programs(0) - 1)
        def _():
            nm2_ref[0, 0] = acc_ref[0, 0]

    return pl.pallas_call(
        body,
        grid=(N // MB,),
        in_specs=[
            pl.BlockSpec((MB, FEAT), lambda i: (i, 0)),
            pl.BlockSpec((MB, 1), lambda i: (i, 0)),
        ],
        out_specs=[
            pl.BlockSpec((1, 1), lambda i: (0, 0)),
            pl.BlockSpec((MB, 1), lambda i: (i, 0)),
        ],
        out_shape=[
            jax.ShapeDtypeStruct((1, 1), jnp.float32),
            jax.ShapeDtypeStruct((N, 1), jnp.float32),
        ],
        scratch_shapes=[pltpu.VMEM((1, 1), jnp.float32)],
        compiler_params=pltpu.CompilerParams(
            dimension_semantics=("arbitrary",)),
    )(x, deg2d)


def _mm_layer1(x, w, dinv2d, nm2):
    nkb, nfb = NKB[0], NFB[0]

    def body(x_ref, w_ref, dinv_ref, nm2_ref, out_ref, acc_ref):
        k = pl.program_id(2)

        @pl.when(k == 0)
        def _():
            acc_ref[...] = jnp.zeros_like(acc_ref)

        xb = x_ref[...]
        col = k * FB + lax.broadcasted_iota(jnp.int32, xb.shape, 1)
        xb = jnp.where(col < FEAT, xb, 0.0)
        acc_ref[...] += jnp.dot(xb, w_ref[...],
                                preferred_element_type=jnp.float32)

        @pl.when(k == nkb - 1)
        def _():
            scale = dinv_ref[...] * lax.rsqrt(nm2_ref[0, 0])
            out_ref[0] = acc_ref[...] * scale

    return pl.pallas_call(
        body,
        grid=(N // MB, nfb, nkb),
        in_specs=[
            pl.BlockSpec((MB, FB), lambda m, f, k: (m, k)),
            pl.BlockSpec((FB, FB), lambda m, f, k: (k, f)),
            pl.BlockSpec((MB, 1), lambda m, f, k: (m, 0)),
            pl.BlockSpec((1, 1), lambda m, f, k: (0, 0)),
        ],
        out_specs=pl.BlockSpec((1, MB, FB), lambda m, f, k: (f, m, 0)),
        out_shape=jax.ShapeDtypeStruct((nfb, N, FB), jnp.float32),
        scratch_shapes=[pltpu.VMEM((MB, FB), jnp.float32)],
        compiler_params=pltpu.CompilerParams(
            dimension_semantics=("parallel", "parallel", "arbitrary")),
    )(x, w, dinv2d, nm2)


def _mm_layer(layer, acc_in, w, b2d, dinv2d):
    nkb, nfb = NKB[layer], NFB[layer]

    def body(x_ref, w_ref, b_ref, dinv_ref, out_ref, acc_ref):
        k = pl.program_id(2)

        @pl.when(k == 0)
        def _():
            acc_ref[...] = jnp.zeros_like(acc_ref)

        dv = dinv_ref[...]
        xb = jnp.maximum(x_ref[0] * dv + b_ref[...], 0.0)
        acc_ref[...] += jnp.dot(xb, w_ref[...],
                                preferred_element_type=jnp.float32)

        @pl.when(k == nkb - 1)
        def _():
            out_ref[0] = acc_ref[...] * dv

    return pl.pallas_call(
        body,
        grid=(N // MB, nfb, nkb),
        in_specs=[
            pl.BlockSpec((1, MB, FB), lambda m, f, k: (k, m, 0)),
            pl.BlockSpec((FB, FB), lambda m, f, k: (k, f)),
            pl.BlockSpec((1, FB), lambda m, f, k: (k, 0)),
            pl.BlockSpec((MB, 1), lambda m, f, k: (m, 0)),
        ],
        out_specs=pl.BlockSpec((1, MB, FB), lambda m, f, k: (f, m, 0)),
        out_shape=jax.ShapeDtypeStruct((nfb, N, FB), jnp.float32),
        scratch_shapes=[pltpu.VMEM((MB, FB), jnp.float32)],
        compiler_params=pltpu.CompilerParams(
            dimension_semantics=("parallel", "parallel", "arbitrary")),
    )(acc_in, w, b2d, dinv2d)


def _final(acc_in, dinv2d, b6):
    def body(acc_ref, dinv_ref, b_ref, out_ref):
        out_ref[...] = acc_ref[0][:, 0:1] * dinv_ref[...] + b_ref[0, 0]

    return pl.pallas_call(
        body,
        grid=(N // MB,),
        in_specs=[
            pl.BlockSpec((1, MB, FB), lambda m: (0, m, 0)),
            pl.BlockSpec((MB, 1), lambda m: (m, 0)),
            pl.BlockSpec((1, 1), lambda m: (0, 0)),
        ],
        out_specs=pl.BlockSpec((MB, 1), lambda m: (m, 0)),
        out_shape=jax.ShapeDtypeStruct((N, 1), jnp.float32),
    )(acc_in, dinv2d, b6)


def kernel(x, edge_index, batch, W1, b1, W2, b2, W3, b3, W4, b4, W5, b5,
           W6, b6):
    del batch
    src2d = edge_index[0].reshape(NCHUNK, CH)
    dst2d = edge_index[1].reshape(NCHUNK, CH)
    ones_n = jnp.ones((N,), jnp.float32)

    deg = _degree_kernel(dst2d, ones_n)
    nm2, dinv2d = _norm_dinv(x, deg.reshape(N, 1))

    Ws = [W1, W2, W3, W4, W5, W6]
    bs = [b1, b2, b3, b4, b5, b6]

    w = jnp.pad(W1, ((0, KPAD[0] - DIMS_K[0]), (0, FPAD[0] - DIMS_F[0])))
    hs = _mm_layer1(x, w, dinv2d, nm2)
    acc = _SCATTER[NFB[0]](hs, src2d, dst2d)

    for l in range(1, 6):
        w = jnp.pad(Ws[l], ((0, KPAD[l] - DIMS_K[l]), (0, FPAD[l] - DIMS_F[l])))
        b2d = jnp.pad(bs[l - 1], (0, KPAD[l] - DIMS_K[l])).reshape(NKB[l], FB)
        hs = _mm_layer(l, acc, w, b2d, dinv2d)
        acc = _SCATTER[NFB[l]](hs, src2d, dst2d)

    out = _final(acc, dinv2d, b6.reshape(1, 1))
    return out.reshape(1, N)


# speedup vs baseline: 2.3120x; 2.3120x over previous
import functools

import jax
import jax.numpy as jnp
from jax import lax
from jax.experimental import pallas as pl
from jax.experimental.pallas import tpu as pltpu
from jax.experimental.pallas import tpu_sc as plsc

N = 10000
E = 160000
FEAT = 3244
MB = 1000
FB = 128
CH = 125
NCHUNK = E // CH
SC_TILES = 16
ROWS_PER_TILE = NCHUNK // SC_TILES

DIMS_K = [FEAT, 2000, 1000, 500, 100, 10]
DIMS_F = [2000, 1000, 500, 100, 10, 1]
KPAD = [3328, 2048, 1024, 512, 128, 128]
FPAD = [2048, 1024, 512, 128, 128, 128]
NKB = [kp // FB for kp in KPAD]
NFB = [fp // FB for fp in FPAD]

@functools.lru_cache(maxsize=None)
def _sc_mesh():
    return plsc.VectorSubcoreMesh(
        core_axis_name="c", subcore_axis_name="s",
        num_cores=2, num_subcores=16)


def _degree_kernel(dst2d, ones_n):
    @functools.partial(
        pl.kernel,
        out_type=jax.ShapeDtypeStruct((N,), jnp.float32),
        mesh=_sc_mesh(),
        scratch_types=[
            pltpu.VMEM_SHARED((N,), jnp.float32),
            pltpu.VMEM((ROWS_PER_TILE, CH), jnp.int32),
            pltpu.VMEM((CH,), jnp.float32),
        ],
    )
    def deg_kernel(dst_hbm, ones_hbm, deg_hbm, deg_sp, dst_v, ones_v):
        c = lax.axis_index("c")
        s = lax.axis_index("s")

        @pl.when(c == 0)
        def _():
            pltpu.sync_copy(dst_hbm.at[pl.ds(s * ROWS_PER_TILE, ROWS_PER_TILE), :],
                            dst_v)
            pltpu.sync_copy(ones_hbm.at[pl.ds(0, CH)], ones_v)

            @pl.when(s == 0)
            def _():
                pltpu.sync_copy(ones_hbm, deg_sp)

            plsc.subcore_barrier()

            def body(j, carry):
                pltpu.sync_copy(ones_v, deg_sp.at[dst_v.at[j]], add=True)
                return carry

            lax.fori_loop(0, ROWS_PER_TILE, body, 0)
            plsc.subcore_barrier()

            @pl.when(s == 0)
            def _():
                pltpu.sync_copy(deg_sp, deg_hbm)

    return deg_kernel(dst2d, ones_n)


@functools.lru_cache(maxsize=None)
def _make_scatter(n_fb):
    @functools.partial(
        pl.kernel,
        out_type=jax.ShapeDtypeStruct((n_fb, N, FB), jnp.float32),
        mesh=_sc_mesh(),
        scratch_types=[
            pltpu.VMEM_SHARED((N, FB), jnp.float32),
            pltpu.VMEM((ROWS_PER_TILE, CH), jnp.int32),
            pltpu.VMEM((ROWS_PER_TILE, CH), jnp.int32),
            pltpu.VMEM((CH, FB), jnp.float32),
            pltpu.SemaphoreType.DMA,
        ],
    )
    def scat_kernel(hs_hbm, src_hbm, dst_hbm, acc_hbm,
                    acc_sp, src_v, dst_v, rows_v, sem):
        c = lax.axis_index("c")
        s = lax.axis_index("s")
        row0 = s * ROWS_PER_TILE
        pltpu.sync_copy(src_hbm.at[pl.ds(row0, ROWS_PER_TILE), :], src_v)
        pltpu.sync_copy(dst_hbm.at[pl.ds(row0, ROWS_PER_TILE), :], dst_v)
        for fb in range(n_fb):
            owner = fb % 2

            @pl.when(c == owner)
            def _(fb=fb):
                @pl.when(s == 0)
                def _():
                    pltpu.sync_copy(hs_hbm.at[fb], acc_sp)

                plsc.subcore_barrier()

                def body(j, carry):
                    pltpu.async_copy(
                        hs_hbm.at[fb].at[src_v.at[j]], rows_v, sem).wait()
                    pltpu.sync_copy(rows_v, acc_sp.at[dst_v.at[j]], add=True)
                    return carry

                lax.fori_loop(0, ROWS_PER_TILE, body, 0)
                plsc.subcore_barrier()

                @pl.when(s == 0)
                def _():
                    pltpu.sync_copy(acc_sp, acc_hbm.at[fb])

                plsc.subcore_barrier()

    return scat_kernel


def _norm_dinv(x, deg2d):
    def body(x_ref, deg_ref, nm2_ref, dinv_ref, acc_ref):
        i = pl.program_id(0)

        @pl.when(i == 0)
        def _():
            acc_ref[...] = jnp.zeros_like(acc_ref)

        xb = x_ref[...]
        acc_ref[...] += jnp.sum(xb * xb).reshape(1, 1)
        dinv_ref[...] = lax.rsqrt(deg_ref[...])

        @pl.when(i == pl.num_programs(0) - 1)
        def _():
            nm2_ref[...] = acc_ref[...]

    return pl.pallas_call(
        body,
        grid=(N // MB,),
        in_specs=[
            pl.BlockSpec((MB, FEAT), lambda i: (i, 0)),
            pl.BlockSpec((MB, 1), lambda i: (i, 0)),
        ],
        out_specs=[
            pl.BlockSpec((1, 1), lambda i: (0, 0)),
            pl.BlockSpec((MB, 1), lambda i: (i, 0)),
        ],
        out_shape=[
            jax.ShapeDtypeStruct((1, 1), jnp.float32),
            jax.ShapeDtypeStruct((N, 1), jnp.float32),
        ],
        scratch_shapes=[pltpu.VMEM((1, 1), jnp.float32)],
        compiler_params=pltpu.CompilerParams(
            dimension_semantics=("arbitrary",)),
    )(x, deg2d)


def _mm_layer1(x, w, dinv2d, nm2):
    nkb, nfb = NKB[0], NFB[0]

    def body(x_ref, w_ref, dinv_ref, nm2_ref, out_ref, acc_ref):
        k = pl.program_id(2)

        @pl.when(k == 0)
        def _():
            acc_ref[...] = jnp.zeros_like(acc_ref)

        xb = x_ref[...]
        col = k * FB + lax.broadcasted_iota(jnp.int32, xb.shape, 1)
        xb = jnp.where(col < FEAT, xb, 0.0)
        acc_ref[...] += jnp.dot(xb, w_ref[...],
                                preferred_element_type=jnp.float32)

        @pl.when(k == nkb - 1)
        def _():
            scale = dinv_ref[...] * lax.rsqrt(nm2_ref[...])
            out_ref[0] = acc_ref[...] * scale

    return pl.pallas_call(
        body,
        grid=(N // MB, nfb, nkb),
        in_specs=[
            pl.BlockSpec((MB, FB), lambda m, f, k: (m, k)),
            pl.BlockSpec((FB, FB), lambda m, f, k: (k, f)),
            pl.BlockSpec((MB, 1), lambda m, f, k: (m, 0)),
            pl.BlockSpec((1, 1), lambda m, f, k: (0, 0)),
        ],
        out_specs=pl.BlockSpec((1, MB, FB), lambda m, f, k: (f, m, 0)),
        out_shape=jax.ShapeDtypeStruct((nfb, N, FB), jnp.float32),
        scratch_shapes=[pltpu.VMEM((MB, FB), jnp.float32)],
        compiler_params=pltpu.CompilerParams(
            dimension_semantics=("parallel", "parallel", "arbitrary")),
    )(x, w, dinv2d, nm2)


def _mm_layer(layer, acc_in, w, b2d, dinv2d):
    nkb, nfb = NKB[layer], NFB[layer]

    def body(x_ref, w_ref, b_ref, dinv_ref, out_ref, acc_ref):
        k = pl.program_id(2)

        @pl.when(k == 0)
        def _():
            acc_ref[...] = jnp.zeros_like(acc_ref)

        dv = dinv_ref[...]
        xb = jnp.maximum(x_ref[0] * dv + b_ref[0], 0.0)
        acc_ref[...] += jnp.dot(xb, w_ref[...],
                                preferred_element_type=jnp.float32)

        @pl.when(k == nkb - 1)
        def _():
            out_ref[0] = acc_ref[...] * dv

    return pl.pallas_call(
        body,
        grid=(N // MB, nfb, nkb),
        in_specs=[
            pl.BlockSpec((1, MB, FB), lambda m, f, k: (k, m, 0)),
            pl.BlockSpec((FB, FB), lambda m, f, k: (k, f)),
            pl.BlockSpec((1, 1, FB), lambda m, f, k: (k, 0, 0)),
            pl.BlockSpec((MB, 1), lambda m, f, k: (m, 0)),
        ],
        out_specs=pl.BlockSpec((1, MB, FB), lambda m, f, k: (f, m, 0)),
        out_shape=jax.ShapeDtypeStruct((nfb, N, FB), jnp.float32),
        scratch_shapes=[pltpu.VMEM((MB, FB), jnp.float32)],
        compiler_params=pltpu.CompilerParams(
            dimension_semantics=("parallel", "parallel", "arbitrary")),
    )(acc_in, w, b2d, dinv2d)


def _final(acc_in, dinv2d, b6):
    def body(acc_ref, dinv_ref, b_ref, out_ref):
        out_ref[...] = acc_ref[0][:, 0:1] * dinv_ref[...] + b_ref[...]

    return pl.pallas_call(
        body,
        grid=(N // MB,),
        in_specs=[
            pl.BlockSpec((1, MB, FB), lambda m: (0, m, 0)),
            pl.BlockSpec((MB, 1), lambda m: (m, 0)),
            pl.BlockSpec((1, 1), lambda m: (0, 0)),
        ],
        out_specs=pl.BlockSpec((MB, 1), lambda m: (m, 0)),
        out_shape=jax.ShapeDtypeStruct((N, 1), jnp.float32),
    )(acc_in, dinv2d, b6)


def kernel(x, edge_index, batch, W1, b1, W2, b2, W3, b3, W4, b4, W5, b5,
           W6, b6):
    del batch
    src2d = edge_index[0].reshape(NCHUNK, CH)
    dst2d = edge_index[1].reshape(NCHUNK, CH)
    ones_n = jnp.ones((N,), jnp.float32)

    deg = _degree_kernel(dst2d, ones_n)
    nm2, dinv2d = _norm_dinv(x, deg.reshape(N, 1))

    Ws = [W1, W2, W3, W4, W5, W6]
    bs = [b1, b2, b3, b4, b5, b6]

    w = jnp.pad(W1, ((0, KPAD[0] - DIMS_K[0]), (0, FPAD[0] - DIMS_F[0])))
    hs = _mm_layer1(x, w, dinv2d, nm2)
    acc = _make_scatter(NFB[0])(hs, src2d, dst2d)

    for l in range(1, 6):
        w = jnp.pad(Ws[l], ((0, KPAD[l] - DIMS_K[l]), (0, FPAD[l] - DIMS_F[l])))
        b2d = jnp.pad(bs[l - 1], (0, KPAD[l] - DIMS_K[l])).reshape(NKB[l], 1, FB)
        hs = _mm_layer(l, acc, w, b2d, dinv2d)
        acc = _make_scatter(NFB[l])(hs, src2d, dst2d)

    out = _final(acc, dinv2d, b6.reshape(1, 1))
    return out.reshape(1, N)


# speedup vs baseline: 5.8667x; 2.5375x over previous
import functools

import jax
import jax.numpy as jnp
from jax import lax
from jax.experimental import pallas as pl
from jax.experimental.pallas import tpu as pltpu
from jax.experimental.pallas import tpu_sc as plsc

N = 10000
E = 160000
FEAT = 3244
MB = 1000
FB = 128
CH = 125
NCHUNK = E // CH
SC_TILES = 16
ROWS_PER_TILE = NCHUNK // SC_TILES

DIMS_K = [FEAT, 2000, 1000, 500, 100, 10]
DIMS_F = [2000, 1000, 500, 100, 10, 1]
KPAD = [3328, 2048, 1024, 512, 128, 128]
FPAD = [2048, 1024, 512, 128, 128, 128]
NKB = [kp // FB for kp in KPAD]
NFB = [fp // FB for fp in FPAD]

@functools.lru_cache(maxsize=None)
def _sc_mesh():
    return plsc.VectorSubcoreMesh(
        core_axis_name="c", subcore_axis_name="s",
        num_cores=2, num_subcores=16)


def _degree_kernel(dst2d, ones_n):
    @functools.partial(
        pl.kernel,
        out_type=jax.ShapeDtypeStruct((N,), jnp.float32),
        mesh=_sc_mesh(),
        scratch_types=[
            pltpu.VMEM_SHARED((N,), jnp.float32),
            pltpu.VMEM((1, CH), jnp.int32),
            pltpu.VMEM((CH,), jnp.float32),
        ],
    )
    def deg_kernel(dst_hbm, ones_hbm, deg_hbm, deg_sp, dst_v, ones_v):
        c = lax.axis_index("c")
        s = lax.axis_index("s")

        @pl.when(c == 0)
        def _():
            row0 = s * ROWS_PER_TILE
            pltpu.sync_copy(ones_hbm.at[pl.ds(0, CH)], ones_v)

            @pl.when(s == 0)
            def _():
                pltpu.sync_copy(ones_hbm, deg_sp)

            plsc.subcore_barrier()

            def body(j, carry):
                pltpu.sync_copy(dst_hbm.at[row0 + j], dst_v)
                pltpu.sync_copy(ones_v, deg_sp.at[dst_v.at[0]], add=True)
                return carry

            lax.fori_loop(0, ROWS_PER_TILE, body, 0)
            plsc.subcore_barrier()

            @pl.when(s == 0)
            def _():
                pltpu.sync_copy(deg_sp, deg_hbm)

    return deg_kernel(dst2d, ones_n)


@functools.lru_cache(maxsize=None)
def _make_scatter(n_fb):
    @functools.partial(
        pl.kernel,
        out_type=jax.ShapeDtypeStruct((n_fb, N, FB), jnp.float32),
        mesh=_sc_mesh(),
        scratch_types=[
            pltpu.VMEM_SHARED((N, FB), jnp.float32),
            pltpu.VMEM((1, CH), jnp.int32),
            pltpu.VMEM((1, CH), jnp.int32),
            pltpu.VMEM((1, CH), jnp.int32),
            pltpu.VMEM((1, CH), jnp.int32),
            pltpu.VMEM((CH, FB), jnp.float32),
            pltpu.VMEM((CH, FB), jnp.float32),
            pltpu.SemaphoreType.DMA,
            pltpu.SemaphoreType.DMA,
            pltpu.SemaphoreType.DMA,
            pltpu.SemaphoreType.DMA,
            pltpu.SemaphoreType.DMA,
            pltpu.SemaphoreType.DMA,
        ],
    )
    def scat_kernel(hs_hbm, src_hbm, dst_hbm, acc_hbm,
                    acc_sp, src_a, dst_a, src_b, dst_b, rows_a, rows_b,
                    isa, ida, isb, idb, g_a, g_b):
        c = lax.axis_index("c")
        s = lax.axis_index("s")
        row0 = s * ROWS_PER_TILE
        nit = ROWS_PER_TILE // 2
        for fb in range(n_fb):
            owner = fb % 2

            @pl.when(c == owner)
            def _(fb=fb):
                @pl.when(s == 0)
                def _():
                    pltpu.sync_copy(hs_hbm.at[fb], acc_sp)

                plsc.subcore_barrier()

                blk = hs_hbm.at[fb]
                pltpu.async_copy(src_hbm.at[row0], src_a, isa)
                pltpu.async_copy(dst_hbm.at[row0], dst_a, ida)
                pltpu.make_async_copy(src_hbm.at[row0], src_a, isa).wait()
                pltpu.async_copy(blk.at[src_a.at[0]], rows_a, g_a)
                pltpu.async_copy(src_hbm.at[row0 + 1], src_b, isb)
                pltpu.async_copy(dst_hbm.at[row0 + 1], dst_b, idb)

                def body(i, carry):
                    ja = row0 + 2 * i
                    jb = ja + 1
                    pltpu.make_async_copy(src_hbm.at[jb], src_b, isb).wait()
                    pltpu.async_copy(blk.at[src_b.at[0]], rows_b, g_b)
                    pltpu.make_async_copy(dst_hbm.at[ja], dst_a, ida).wait()
                    pltpu.make_async_copy(
                        blk.at[src_a.at[0]], rows_a, g_a).wait()
                    pltpu.sync_copy(rows_a, acc_sp.at[dst_a.at[0]], add=True)

                    @pl.when(i < nit - 1)
                    def _():
                        pltpu.async_copy(src_hbm.at[ja + 2], src_a, isa)
                        pltpu.async_copy(dst_hbm.at[ja + 2], dst_a, ida)
                        pltpu.make_async_copy(
                            src_hbm.at[ja + 2], src_a, isa).wait()
                        pltpu.async_copy(blk.at[src_a.at[0]], rows_a, g_a)

                    pltpu.make_async_copy(dst_hbm.at[jb], dst_b, idb).wait()
                    pltpu.make_async_copy(
                        blk.at[src_b.at[0]], rows_b, g_b).wait()
                    pltpu.sync_copy(rows_b, acc_sp.at[dst_b.at[0]], add=True)

                    @pl.when(i < nit - 1)
                    def _():
                        pltpu.async_copy(src_hbm.at[jb + 2], src_b, isb)
                        pltpu.async_copy(dst_hbm.at[jb + 2], dst_b, idb)

                    return carry

                lax.fori_loop(0, nit, body, 0)
                plsc.subcore_barrier()

                @pl.when(s == 0)
                def _():
                    pltpu.sync_copy(acc_sp, acc_hbm.at[fb])

                plsc.subcore_barrier()

    return scat_kernel


def _norm_dinv(x, deg2d):
    def body(x_ref, deg_ref, nm2_ref, dinv_ref, acc_ref):
        i = pl.program_id(0)

        @pl.when(i == 0)
        def _():
            acc_ref[...] = jnp.zeros_like(acc_ref)

        xb = x_ref[...]
        acc_ref[...] += jnp.sum(xb * xb).reshape(1, 1)
        dinv_ref[...] = lax.rsqrt(deg_ref[...])

        @pl.when(i == pl.num_programs(0) - 1)
        def _():
            nm2_ref[...] = acc_ref[...]

    return pl.pallas_call(
        body,
        grid=(N // MB,),
        in_specs=[
            pl.BlockSpec((MB, FEAT), lambda i: (i, 0)),
            pl.BlockSpec((MB, 1), lambda i: (i, 0)),
        ],
        out_specs=[
            pl.BlockSpec((1, 1), lambda i: (0, 0)),
            pl.BlockSpec((MB, 1), lambda i: (i, 0)),
        ],
        out_shape=[
            jax.ShapeDtypeStruct((1, 1), jnp.float32),
            jax.ShapeDtypeStruct((N, 1), jnp.float32),
        ],
        scratch_shapes=[pltpu.VMEM((1, 1), jnp.float32)],
        compiler_params=pltpu.CompilerParams(
            dimension_semantics=("arbitrary",)),
    )(x, deg2d)


def _mm_layer1(x, w, dinv2d, nm2):
    nkb, nfb = NKB[0], NFB[0]
    fpad = FPAD[0]

    def body(x_ref, w_ref, dinv_ref, nm2_ref, out_ref, acc_ref):
        k = pl.program_id(1)

        @pl.when(k == 0)
        def _():
            acc_ref[...] = jnp.zeros_like(acc_ref)

        xb = x_ref[...]
        col = k * FB + lax.broadcasted_iota(jnp.int32, xb.shape, 1)
        xb = jnp.where(col < FEAT, xb, 0.0)
        acc_ref[...] += jnp.dot(xb, w_ref[...],
                                preferred_element_type=jnp.float32)

        @pl.when(k == nkb - 1)
        def _():
            scale = dinv_ref[...] * lax.rsqrt(nm2_ref[...])
            for f in range(nfb):
                out_ref[f] = acc_ref[:, f * FB:(f + 1) * FB] * scale

    return pl.pallas_call(
        body,
        grid=(N // MB, nkb),
        in_specs=[
            pl.BlockSpec((MB, FB), lambda m, k: (m, k)),
            pl.BlockSpec((FB, fpad), lambda m, k: (k, 0)),
            pl.BlockSpec((MB, 1), lambda m, k: (m, 0)),
            pl.BlockSpec((1, 1), lambda m, k: (0, 0)),
        ],
        out_specs=pl.BlockSpec((nfb, MB, FB), lambda m, k: (0, m, 0)),
        out_shape=jax.ShapeDtypeStruct((nfb, N, FB), jnp.float32),
        scratch_shapes=[pltpu.VMEM((MB, fpad), jnp.float32)],
        compiler_params=pltpu.CompilerParams(
            dimension_semantics=("parallel", "arbitrary")),
    )(x, w, dinv2d, nm2)


def _mm_layer(layer, acc_in, w, b2d, dinv2d):
    nkb, nfb = NKB[layer], NFB[layer]
    fpad = FPAD[layer]

    def body(x_ref, w_ref, b_ref, dinv_ref, out_ref, acc_ref):
        k = pl.program_id(1)

        @pl.when(k == 0)
        def _():
            acc_ref[...] = jnp.zeros_like(acc_ref)

        dv = dinv_ref[...]
        xb = jnp.maximum(x_ref[0] * dv + b_ref[0], 0.0)
        acc_ref[...] += jnp.dot(xb, w_ref[...],
                                preferred_element_type=jnp.float32)

        @pl.when(k == nkb - 1)
        def _():
            for f in range(nfb):
                out_ref[f] = acc_ref[:, f * FB:(f + 1) * FB] * dv

    return pl.pallas_call(
        body,
        grid=(N // MB, nkb),
        in_specs=[
            pl.BlockSpec((1, MB, FB), lambda m, k: (k, m, 0)),
            pl.BlockSpec((FB, fpad), lambda m, k: (k, 0)),
            pl.BlockSpec((1, 1, FB), lambda m, k: (k, 0, 0)),
            pl.BlockSpec((MB, 1), lambda m, k: (m, 0)),
        ],
        out_specs=pl.BlockSpec((nfb, MB, FB), lambda m, k: (0, m, 0)),
        out_shape=jax.ShapeDtypeStruct((nfb, N, FB), jnp.float32),
        scratch_shapes=[pltpu.VMEM((MB, fpad), jnp.float32)],
        compiler_params=pltpu.CompilerParams(
            dimension_semantics=("parallel", "arbitrary")),
    )(acc_in, w, b2d, dinv2d)


def _final(acc_in, dinv2d, b6):
    def body(acc_ref, dinv_ref, b_ref, out_ref):
        out_ref[...] = acc_ref[0][:, 0:1] * dinv_ref[...] + b_ref[...]

    return pl.pallas_call(
        body,
        grid=(N // MB,),
        in_specs=[
            pl.BlockSpec((1, MB, FB), lambda m: (0, m, 0)),
            pl.BlockSpec((MB, 1), lambda m: (m, 0)),
            pl.BlockSpec((1, 1), lambda m: (0, 0)),
        ],
        out_specs=pl.BlockSpec((MB, 1), lambda m: (m, 0)),
        out_shape=jax.ShapeDtypeStruct((N, 1), jnp.float32),
    )(acc_in, dinv2d, b6)


def kernel(x, edge_index, batch, W1, b1, W2, b2, W3, b3, W4, b4, W5, b5,
           W6, b6):
    del batch
    src2d = edge_index[0].reshape(NCHUNK, 1, CH)
    dst2d = edge_index[1].reshape(NCHUNK, 1, CH)
    ones_n = jnp.ones((N,), jnp.float32)

    deg = _degree_kernel(dst2d, ones_n)
    nm2, dinv2d = _norm_dinv(x, deg.reshape(N, 1))

    Ws = [W1, W2, W3, W4, W5, W6]
    bs = [b1, b2, b3, b4, b5, b6]

    w = jnp.pad(W1, ((0, KPAD[0] - DIMS_K[0]), (0, FPAD[0] - DIMS_F[0])))
    hs = _mm_layer1(x, w, dinv2d, nm2)
    acc = _make_scatter(NFB[0])(hs, src2d, dst2d)

    for l in range(1, 6):
        w = jnp.pad(Ws[l], ((0, KPAD[l] - DIMS_K[l]), (0, FPAD[l] - DIMS_F[l])))
        b2d = jnp.pad(bs[l - 1], (0, KPAD[l] - DIMS_K[l])).reshape(NKB[l], 1, FB)
        hs = _mm_layer(l, acc, w, b2d, dinv2d)
        acc = _make_scatter(NFB[l])(hs, src2d, dst2d)

    out = _final(acc, dinv2d, b6.reshape(1, 1))
    return out.reshape(1, N)


# speedup vs baseline: 6.1407x; 1.0467x over previous
import functools

import jax
import jax.numpy as jnp
from jax import lax
from jax.experimental import pallas as pl
from jax.experimental.pallas import tpu as pltpu
from jax.experimental.pallas import tpu_sc as plsc

N = 10000
E = 160000
FEAT = 3244
MB = 1000
FB = 128
CH = 125
NCHUNK = E // CH
SC_TILES = 16
ROWS_PER_TILE = NCHUNK // SC_TILES

DIMS_K = [FEAT, 2000, 1000, 500, 100, 10]
DIMS_F = [2000, 1000, 500, 100, 10, 1]
KPAD = [3328, 2048, 1024, 512, 128, 128]
FPAD = [2048, 1024, 512, 128, 128, 128]
NKB = [kp // FB for kp in KPAD]
NFB = [fp // FB for fp in FPAD]

@functools.lru_cache(maxsize=None)
def _sc_mesh():
    return plsc.VectorSubcoreMesh(
        core_axis_name="c", subcore_axis_name="s",
        num_cores=2, num_subcores=16)


def _degree_kernel(dst2d, init2):
    @functools.partial(
        pl.kernel,
        out_type=jax.ShapeDtypeStruct((2, N), jnp.float32),
        mesh=_sc_mesh(),
        scratch_types=[
            pltpu.VMEM_SHARED((N,), jnp.float32),
            pltpu.VMEM((1, CH), jnp.int32),
            pltpu.VMEM((CH,), jnp.float32),
        ],
    )
    def deg_kernel(dst_hbm, init_hbm, deg_hbm, deg_sp, dst_v, ones_v):
        c = lax.axis_index("c")
        s = lax.axis_index("s")
        rows = ROWS_PER_TILE // 2
        row0 = (c * SC_TILES + s) * rows
        pltpu.sync_copy(init_hbm.at[0, pl.ds(0, CH)], ones_v)

        @pl.when(s == 0)
        def _():
            pltpu.sync_copy(init_hbm.at[c], deg_sp)

        plsc.subcore_barrier()

        def body(j, carry):
            pltpu.sync_copy(dst_hbm.at[row0 + j], dst_v)
            pltpu.sync_copy(ones_v, deg_sp.at[dst_v.at[0]], add=True)
            return carry

        lax.fori_loop(0, rows, body, 0)
        plsc.subcore_barrier()

        @pl.when(s == 0)
        def _():
            pltpu.sync_copy(deg_sp, deg_hbm.at[c])

    return deg_kernel(dst2d, init2)


def _edge_pipeline(blk, acc_sp, row0, nit, src_hbm, dst_hbm,
                   src_a, dst_a, src_b, dst_b, rows_a, rows_b,
                   isa, ida, isb, idb, g_a, g_b):
    pltpu.async_copy(src_hbm.at[row0], src_a, isa)
    pltpu.async_copy(dst_hbm.at[row0], dst_a, ida)
    pltpu.make_async_copy(src_hbm.at[row0], src_a, isa).wait()
    pltpu.async_copy(blk.at[src_a.at[0]], rows_a, g_a)
    pltpu.async_copy(src_hbm.at[row0 + 1], src_b, isb)
    pltpu.async_copy(dst_hbm.at[row0 + 1], dst_b, idb)

    def body(i, carry):
        ja = row0 + 2 * i
        jb = ja + 1
        pltpu.make_async_copy(src_hbm.at[jb], src_b, isb).wait()
        pltpu.async_copy(blk.at[src_b.at[0]], rows_b, g_b)
        pltpu.make_async_copy(dst_hbm.at[ja], dst_a, ida).wait()
        pltpu.make_async_copy(blk.at[src_a.at[0]], rows_a, g_a).wait()
        pltpu.sync_copy(rows_a, acc_sp.at[dst_a.at[0]], add=True)

        @pl.when(i < nit - 1)
        def _():
            pltpu.async_copy(src_hbm.at[ja + 2], src_a, isa)
            pltpu.async_copy(dst_hbm.at[ja + 2], dst_a, ida)
            pltpu.make_async_copy(src_hbm.at[ja + 2], src_a, isa).wait()
            pltpu.async_copy(blk.at[src_a.at[0]], rows_a, g_a)

        pltpu.make_async_copy(dst_hbm.at[jb], dst_b, idb).wait()
        pltpu.make_async_copy(blk.at[src_b.at[0]], rows_b, g_b).wait()
        pltpu.sync_copy(rows_b, acc_sp.at[dst_b.at[0]], add=True)

        @pl.when(i < nit - 1)
        def _():
            pltpu.async_copy(src_hbm.at[jb + 2], src_b, isb)
            pltpu.async_copy(dst_hbm.at[jb + 2], dst_b, idb)

        return carry

    lax.fori_loop(0, nit, body, 0)


@functools.lru_cache(maxsize=None)
def _make_scatter(n_fb):
    n_out = max(n_fb, 2)

    @functools.partial(
        pl.kernel,
        out_type=jax.ShapeDtypeStruct((n_out, N, FB), jnp.float32),
        mesh=_sc_mesh(),
        scratch_types=[
            pltpu.VMEM_SHARED((N, FB), jnp.float32),
            pltpu.VMEM((1, CH), jnp.int32),
            pltpu.VMEM((1, CH), jnp.int32),
            pltpu.VMEM((1, CH), jnp.int32),
            pltpu.VMEM((1, CH), jnp.int32),
            pltpu.VMEM((CH, FB), jnp.float32),
            pltpu.VMEM((CH, FB), jnp.float32),
            pltpu.SemaphoreType.DMA,
            pltpu.SemaphoreType.DMA,
            pltpu.SemaphoreType.DMA,
            pltpu.SemaphoreType.DMA,
            pltpu.SemaphoreType.DMA,
            pltpu.SemaphoreType.DMA,
        ],
    )
    def scat_kernel(hs_hbm, src_hbm, dst_hbm, acc_hbm,
                    acc_sp, src_a, dst_a, src_b, dst_b, rows_a, rows_b,
                    isa, ida, isb, idb, g_a, g_b):
        c = lax.axis_index("c")
        s = lax.axis_index("s")
        bufs = (src_a, dst_a, src_b, dst_b, rows_a, rows_b,
                isa, ida, isb, idb, g_a, g_b)
        if n_fb == 1:
            rows = ROWS_PER_TILE // 2
            row0 = (c * SC_TILES + s) * rows

            @pl.when(s == 0)
            def _():
                pltpu.sync_copy(hs_hbm.at[c], acc_sp)

            plsc.subcore_barrier()
            _edge_pipeline(hs_hbm.at[0], acc_sp, row0, rows // 2,
                           src_hbm, dst_hbm, *bufs)
            plsc.subcore_barrier()

            @pl.when(s == 0)
            def _():
                pltpu.sync_copy(acc_sp, acc_hbm.at[c])
        else:
            row0 = s * ROWS_PER_TILE
            for fb in range(n_fb):
                owner = fb % 2

                @pl.when(c == owner)
                def _(fb=fb):
                    @pl.when(s == 0)
                    def _():
                        pltpu.sync_copy(hs_hbm.at[fb], acc_sp)

                    plsc.subcore_barrier()
                    _edge_pipeline(hs_hbm.at[fb], acc_sp, row0,
                                   ROWS_PER_TILE // 2,
                                   src_hbm, dst_hbm, *bufs)
                    plsc.subcore_barrier()

                    @pl.when(s == 0)
                    def _():
                        pltpu.sync_copy(acc_sp, acc_hbm.at[fb])

                    plsc.subcore_barrier()

    return scat_kernel


def _norm_dinv(x, deg3d):
    def body(x_ref, deg_ref, nm2_ref, dinv_ref, acc_ref):
        i = pl.program_id(0)

        @pl.when(i == 0)
        def _():
            acc_ref[...] = jnp.zeros_like(acc_ref)

        xb = x_ref[...]
        acc_ref[...] += jnp.sum(xb * xb).reshape(1, 1)
        dinv_ref[...] = lax.rsqrt(deg_ref[0] + deg_ref[1])

        @pl.when(i == pl.num_programs(0) - 1)
        def _():
            nm2_ref[...] = acc_ref[...]

    return pl.pallas_call(
        body,
        grid=(N // MB,),
        in_specs=[
            pl.BlockSpec((MB, FEAT), lambda i: (i, 0)),
            pl.BlockSpec((2, MB, 1), lambda i: (0, i, 0)),
        ],
        out_specs=[
            pl.BlockSpec((1, 1), lambda i: (0, 0)),
            pl.BlockSpec((MB, 1), lambda i: (i, 0)),
        ],
        out_shape=[
            jax.ShapeDtypeStruct((1, 1), jnp.float32),
            jax.ShapeDtypeStruct((N, 1), jnp.float32),
        ],
        scratch_shapes=[pltpu.VMEM((1, 1), jnp.float32)],
        compiler_params=pltpu.CompilerParams(
            dimension_semantics=("arbitrary",)),
    )(x, deg3d)


def _mm_layer1(x, w, dinv2d, nm2):
    nkb, nfb = NKB[0], NFB[0]
    fpad = FPAD[0]

    def body(x_ref, w_ref, dinv_ref, nm2_ref, out_ref, acc_ref):
        k = pl.program_id(1)

        @pl.when(k == 0)
        def _():
            acc_ref[...] = jnp.zeros_like(acc_ref)

        xb = x_ref[...]
        col = k * FB + lax.broadcasted_iota(jnp.int32, xb.shape, 1)
        xb = jnp.where(col < FEAT, xb, 0.0)
        acc_ref[...] += jnp.dot(xb, w_ref[...],
                                preferred_element_type=jnp.float32)

        @pl.when(k == nkb - 1)
        def _():
            scale = dinv_ref[...] * lax.rsqrt(nm2_ref[...])
            for f in range(nfb):
                out_ref[f] = acc_ref[:, f * FB:(f + 1) * FB] * scale

    return pl.pallas_call(
        body,
        grid=(N // MB, nkb),
        in_specs=[
            pl.BlockSpec((MB, FB), lambda m, k: (m, k)),
            pl.BlockSpec((FB, fpad), lambda m, k: (k, 0)),
            pl.BlockSpec((MB, 1), lambda m, k: (m, 0)),
            pl.BlockSpec((1, 1), lambda m, k: (0, 0)),
        ],
        out_specs=pl.BlockSpec((nfb, MB, FB), lambda m, k: (0, m, 0)),
        out_shape=jax.ShapeDtypeStruct((nfb, N, FB), jnp.float32),
        scratch_shapes=[pltpu.VMEM((MB, fpad), jnp.float32)],
        compiler_params=pltpu.CompilerParams(
            dimension_semantics=("parallel", "arbitrary")),
    )(x, w, dinv2d, nm2)


def _mm_layer(layer, acc_in, w, b2d, dinv2d):
    nkb, nfb = NKB[layer], NFB[layer]
    fpad = FPAD[layer]
    parts_in = NFB[layer - 1] == 1
    parts_out = nfb == 1

    def body(x_ref, w_ref, b_ref, dinv_ref, out_ref, acc_ref):
        k = pl.program_id(1)

        @pl.when(k == 0)
        def _():
            acc_ref[...] = jnp.zeros_like(acc_ref)

        dv = dinv_ref[...]
        xin = x_ref[0] + x_ref[1] if parts_in else x_ref[0]
        xb = jnp.maximum(xin * dv + b_ref[0], 0.0)
        acc_ref[...] += jnp.dot(xb, w_ref[...],
                                preferred_element_type=jnp.float32)

        @pl.when(k == nkb - 1)
        def _():
            if parts_out:
                out_ref[0] = acc_ref[...] * dv
                out_ref[1] = jnp.zeros_like(acc_ref)
            else:
                for f in range(nfb):
                    out_ref[f] = acc_ref[:, f * FB:(f + 1) * FB] * dv

    in_blk = (2, MB, FB) if parts_in else (1, MB, FB)
    in_map = (lambda m, k: (0, m, 0)) if parts_in else (lambda m, k: (k, m, 0))
    n_out = max(nfb, 2) if parts_out else nfb
    return pl.pallas_call(
        body,
        grid=(N // MB, nkb),
        in_specs=[
            pl.BlockSpec(in_blk, in_map),
            pl.BlockSpec((FB, fpad), lambda m, k: (k, 0)),
            pl.BlockSpec((1, 1, FB), lambda m, k: (k, 0, 0)),
            pl.BlockSpec((MB, 1), lambda m, k: (m, 0)),
        ],
        out_specs=pl.BlockSpec((n_out, MB, FB), lambda m, k: (0, m, 0)),
        out_shape=jax.ShapeDtypeStruct((n_out, N, FB), jnp.float32),
        scratch_shapes=[pltpu.VMEM((MB, fpad), jnp.float32)],
        compiler_params=pltpu.CompilerParams(
            dimension_semantics=("parallel", "arbitrary")),
    )(acc_in, w, b2d, dinv2d)


def _final(acc_in, dinv2d, b6):
    def body(acc_ref, dinv_ref, b_ref, out_ref):
        a = acc_ref[0] + acc_ref[1]
        out_ref[...] = a[:, 0:1] * dinv_ref[...] + b_ref[...]

    return pl.pallas_call(
        body,
        grid=(N // MB,),
        in_specs=[
            pl.BlockSpec((2, MB, FB), lambda m: (0, m, 0)),
            pl.BlockSpec((MB, 1), lambda m: (m, 0)),
            pl.BlockSpec((1, 1), lambda m: (0, 0)),
        ],
        out_specs=pl.BlockSpec((MB, 1), lambda m: (m, 0)),
        out_shape=jax.ShapeDtypeStruct((N, 1), jnp.float32),
    )(acc_in, dinv2d, b6)


def kernel(x, edge_index, batch, W1, b1, W2, b2, W3, b3, W4, b4, W5, b5,
           W6, b6):
    del batch
    src2d = edge_index[0].reshape(NCHUNK, 1, CH)
    dst2d = edge_index[1].reshape(NCHUNK, 1, CH)
    init2 = jnp.stack([jnp.ones((N,), jnp.float32),
                       jnp.zeros((N,), jnp.float32)])

    deg = _degree_kernel(dst2d, init2)
    nm2, dinv2d = _norm_dinv(x, deg.reshape(2, N, 1))

    Ws = [W1, W2, W3, W4, W5, W6]
    bs = [b1, b2, b3, b4, b5, b6]

    w = jnp.pad(W1, ((0, KPAD[0] - DIMS_K[0]), (0, FPAD[0] - DIMS_F[0])))
    hs = _mm_layer1(x, w, dinv2d, nm2)
    acc = _make_scatter(NFB[0])(hs, src2d, dst2d)

    for l in range(1, 6):
        w = jnp.pad(Ws[l], ((0, KPAD[l] - DIMS_K[l]), (0, FPAD[l] - DIMS_F[l])))
        b2d = jnp.pad(bs[l - 1], (0, KPAD[l] - DIMS_K[l])).reshape(NKB[l], 1, FB)
        hs = _mm_layer(l, acc, w, b2d, dinv2d)
        acc = _make_scatter(NFB[l])(hs, src2d, dst2d)

    out = _final(acc, dinv2d, b6.reshape(1, 1))
    return out.reshape(1, N)


# speedup vs baseline: 7.0241x; 1.1439x over previous
import functools

import jax
import jax.numpy as jnp
from jax import lax
from jax.experimental import pallas as pl
from jax.experimental.pallas import tpu as pltpu
from jax.experimental.pallas import tpu_sc as plsc

N = 10000
E = 160000
FEAT = 3244
MB = 1000
FB = 128
CH = 125
NCHUNK = E // CH
SC_TILES = 16
ROWS_PER_TILE = NCHUNK // SC_TILES

DIMS_K = [FEAT, 2000, 1000, 500, 100, 10]
DIMS_F = [2000, 1000, 500, 100, 10, 1]
KPAD = [3328, 2048, 1024, 512, 128, 128]
FPAD = [2048, 1024, 512, 128, 128, 128]
NKB = [kp // FB for kp in KPAD]
NFB = [fp // FB for fp in FPAD]

@functools.lru_cache(maxsize=None)
def _sc_mesh():
    return plsc.VectorSubcoreMesh(
        core_axis_name="c", subcore_axis_name="s",
        num_cores=2, num_subcores=16)


def _degree_kernel(dst2d, init2):
    @functools.partial(
        pl.kernel,
        out_type=jax.ShapeDtypeStruct((2, N), jnp.float32),
        mesh=_sc_mesh(),
        scratch_types=[
            pltpu.VMEM_SHARED((N,), jnp.float32),
            pltpu.VMEM((1, CH), jnp.int32),
            pltpu.VMEM((CH,), jnp.float32),
        ],
    )
    def deg_kernel(dst_hbm, init_hbm, deg_hbm, deg_sp, dst_v, ones_v):
        c = lax.axis_index("c")
        s = lax.axis_index("s")
        rows = ROWS_PER_TILE // 2
        row0 = (c * SC_TILES + s) * rows
        pltpu.sync_copy(init_hbm.at[0, pl.ds(0, CH)], ones_v)

        @pl.when(s == 0)
        def _():
            pltpu.sync_copy(init_hbm.at[c], deg_sp)

        plsc.subcore_barrier()

        def body(j, carry):
            pltpu.sync_copy(dst_hbm.at[row0 + j], dst_v)
            pltpu.sync_copy(ones_v, deg_sp.at[dst_v.at[0]], add=True)
            return carry

        lax.fori_loop(0, rows, body, 0)
        plsc.subcore_barrier()

        @pl.when(s == 0)
        def _():
            pltpu.sync_copy(deg_sp, deg_hbm.at[c])

    return deg_kernel(dst2d, init2)


def _edge_pipeline(blk, acc_sp, row0, nchunks, src_hbm, dst_hbm,
                   srcb, dstb, rows, isem, gsem, ssem):
    nit = nchunks // 6
    nfull = nit * 6

    def _idx(c, xs):
        pltpu.async_copy(src_hbm.at[c], srcb[xs], isem[xs])
        pltpu.async_copy(dst_hbm.at[c], dstb[xs], isem[xs])

    def _wait_idx(c, xs):
        pltpu.make_async_copy(src_hbm.at[c], srcb[xs], isem[xs]).wait()
        pltpu.make_async_copy(dst_hbm.at[c], dstb[xs], isem[xs]).wait()

    def _fire_gather(rs, xs):
        pltpu.async_copy(blk.at[srcb[xs].at[0]], rows[rs], gsem[rs])

    def _wait_gather(rs, xs):
        pltpu.make_async_copy(blk.at[srcb[xs].at[0]], rows[rs],
                              gsem[rs]).wait()

    def _fire_scatter(rs, xs):
        pltpu.async_copy(rows[rs], acc_sp.at[dstb[xs].at[0]], ssem[rs],
                         add=True)

    def _wait_scatter(rs, xs):
        pltpu.make_async_copy(rows[rs], acc_sp.at[dstb[xs].at[0]],
                              ssem[rs]).wait()

    for t in range(5):
        _idx(row0 + t, t)
    _wait_idx(row0, 0)
    _fire_gather(0, 0)
    _wait_idx(row0 + 1, 1)
    _fire_gather(1, 1)

    def body(i, carry):
        j = row0 + 6 * i
        for t in range(6):
            c = j + t
            rs, xs = t % 3, t
            rs2, xs2 = (t + 2) % 3, (t + 2) % 6
            xs_prev = (t + 5) % 6

            def _advance(c=c, rs2=rs2, xs2=xs2, xs_prev=xs_prev,
                         skip_wait=False):
                if not skip_wait:
                    _wait_scatter(rs2, xs_prev)
                _wait_idx(c + 2, xs2)
                _fire_gather(rs2, xs2)

            if t == 0:
                @pl.when(i > 0)
                def _():
                    _advance()

                @pl.when(i == 0)
                def _():
                    _advance(skip_wait=True)

                _idx(c + 5, xs_prev)
            elif t >= 4:
                @pl.when(i < nit - 1)
                def _():
                    _advance()
                    _idx(c + 5, xs_prev)
            else:
                _advance()

                @pl.when(i < nit - 1)
                def _():
                    _idx(c + 5, xs_prev)

            _wait_gather(rs, xs)
            _fire_scatter(rs, xs)
        return carry

    lax.fori_loop(0, nit, body, 0)

    for k in range(3):
        _wait_scatter(k, 3 + k)

    for u in range(nchunks - nfull):
        c = row0 + nfull + u
        _idx(c, u)
        _wait_idx(c, u)
        _fire_gather(u % 3, u)
        _wait_gather(u % 3, u)
        pltpu.sync_copy(rows[u % 3], acc_sp.at[dstb[u].at[0]], add=True)


@functools.lru_cache(maxsize=None)
def _make_scatter(n_fb):
    n_out = max(n_fb, 2)

    @functools.partial(
        pl.kernel,
        out_type=jax.ShapeDtypeStruct((n_out, N, FB), jnp.float32),
        mesh=_sc_mesh(),
        scratch_types=(
            [pltpu.VMEM_SHARED((N, FB), jnp.float32)]
            + [pltpu.VMEM((1, CH), jnp.int32)] * 12
            + [pltpu.VMEM((CH, FB), jnp.float32)] * 3
            + [pltpu.SemaphoreType.DMA] * 12
        ),
    )
    def scat_kernel(hs_hbm, src_hbm, dst_hbm, acc_hbm, acc_sp, *sc):
        c = lax.axis_index("c")
        s = lax.axis_index("s")
        bufs = (list(sc[0:6]), list(sc[6:12]), list(sc[12:15]),
                list(sc[15:21]), list(sc[21:24]), list(sc[24:27]))
        if n_fb == 1:
            rows = ROWS_PER_TILE // 2
            row0 = (c * SC_TILES + s) * rows

            @pl.when(s == 0)
            def _():
                pltpu.sync_copy(hs_hbm.at[c], acc_sp)

            plsc.subcore_barrier()
            _edge_pipeline(hs_hbm.at[0], acc_sp, row0, rows,
                           src_hbm, dst_hbm, *bufs)
            plsc.subcore_barrier()

            @pl.when(s == 0)
            def _():
                pltpu.sync_copy(acc_sp, acc_hbm.at[c])
        else:
            row0 = s * ROWS_PER_TILE
            for fb in range(n_fb):
                owner = fb % 2

                @pl.when(c == owner)
                def _(fb=fb):
                    @pl.when(s == 0)
                    def _():
                        pltpu.sync_copy(hs_hbm.at[fb], acc_sp)

                    plsc.subcore_barrier()
                    _edge_pipeline(hs_hbm.at[fb], acc_sp, row0,
                                   ROWS_PER_TILE,
                                   src_hbm, dst_hbm, *bufs)
                    plsc.subcore_barrier()

                    @pl.when(s == 0)
                    def _():
                        pltpu.sync_copy(acc_sp, acc_hbm.at[fb])

                    plsc.subcore_barrier()

    return scat_kernel


def _norm_dinv(x, deg3d):
    def body(x_ref, deg_ref, nm2_ref, dinv_ref, acc_ref):
        i = pl.program_id(0)

        @pl.when(i == 0)
        def _():
            acc_ref[...] = jnp.zeros_like(acc_ref)

        xb = x_ref[...]
        acc_ref[...] += jnp.sum(xb * xb).reshape(1, 1)
        dinv_ref[...] = lax.rsqrt(deg_ref[0] + deg_ref[1])

        @pl.when(i == pl.num_programs(0) - 1)
        def _():
            nm2_ref[...] = acc_ref[...]

    return pl.pallas_call(
        body,
        grid=(N // MB,),
        in_specs=[
            pl.BlockSpec((MB, FEAT), lambda i: (i, 0)),
            pl.BlockSpec((2, MB, 1), lambda i: (0, i, 0)),
        ],
        out_specs=[
            pl.BlockSpec((1, 1), lambda i: (0, 0)),
            pl.BlockSpec((MB, 1), lambda i: (i, 0)),
        ],
        out_shape=[
            jax.ShapeDtypeStruct((1, 1), jnp.float32),
            jax.ShapeDtypeStruct((N, 1), jnp.float32),
        ],
        scratch_shapes=[pltpu.VMEM((1, 1), jnp.float32)],
        compiler_params=pltpu.CompilerParams(
            dimension_semantics=("arbitrary",)),
    )(x, deg3d)


def _mm_layer1(x, w, dinv2d, nm2):
    nkb, nfb = NKB[0], NFB[0]
    fpad = FPAD[0]

    def body(x_ref, w_ref, dinv_ref, nm2_ref, out_ref, acc_ref):
        k = pl.program_id(1)

        @pl.when(k == 0)
        def _():
            acc_ref[...] = jnp.zeros_like(acc_ref)

        xb = x_ref[...]
        col = k * FB + lax.broadcasted_iota(jnp.int32, xb.shape, 1)
        xb = jnp.where(col < FEAT, xb, 0.0)
        acc_ref[...] += jnp.dot(xb, w_ref[...],
                                preferred_element_type=jnp.float32)

        @pl.when(k == nkb - 1)
        def _():
            scale = dinv_ref[...] * lax.rsqrt(nm2_ref[...])
            for f in range(nfb):
                out_ref[f] = acc_ref[:, f * FB:(f + 1) * FB] * scale

    return pl.pallas_call(
        body,
        grid=(N // MB, nkb),
        in_specs=[
            pl.BlockSpec((MB, FB), lambda m, k: (m, k)),
            pl.BlockSpec((FB, fpad), lambda m, k: (k, 0)),
            pl.BlockSpec((MB, 1), lambda m, k: (m, 0)),
            pl.BlockSpec((1, 1), lambda m, k: (0, 0)),
        ],
        out_specs=pl.BlockSpec((nfb, MB, FB), lambda m, k: (0, m, 0)),
        out_shape=jax.ShapeDtypeStruct((nfb, N, FB), jnp.float32),
        scratch_shapes=[pltpu.VMEM((MB, fpad), jnp.float32)],
        compiler_params=pltpu.CompilerParams(
            dimension_semantics=("parallel", "arbitrary")),
    )(x, w, dinv2d, nm2)


def _mm_layer(layer, acc_in, w, b2d, dinv2d):
    nkb, nfb = NKB[layer], NFB[layer]
    fpad = FPAD[layer]
    parts_in = NFB[layer - 1] == 1
    parts_out = nfb == 1

    def body(x_ref, w_ref, b_ref, dinv_ref, out_ref, acc_ref):
        k = pl.program_id(1)

        @pl.when(k == 0)
        def _():
            acc_ref[...] = jnp.zeros_like(acc_ref)

        dv = dinv_ref[...]
        xin = x_ref[0] + x_ref[1] if parts_in else x_ref[0]
        xb = jnp.maximum(xin * dv + b_ref[0], 0.0)
        acc_ref[...] += jnp.dot(xb, w_ref[...],
                                preferred_element_type=jnp.float32)

        @pl.when(k == nkb - 1)
        def _():
            if parts_out:
                out_ref[0] = acc_ref[...] * dv
                out_ref[1] = jnp.zeros_like(acc_ref)
            else:
                for f in range(nfb):
                    out_ref[f] = acc_ref[:, f * FB:(f + 1) * FB] * dv

    in_blk = (2, MB, FB) if parts_in else (1, MB, FB)
    in_map = (lambda m, k: (0, m, 0)) if parts_in else (lambda m, k: (k, m, 0))
    n_out = max(nfb, 2) if parts_out else nfb
    return pl.pallas_call(
        body,
        grid=(N // MB, nkb),
        in_specs=[
            pl.BlockSpec(in_blk, in_map),
            pl.BlockSpec((FB, fpad), lambda m, k: (k, 0)),
            pl.BlockSpec((1, 1, FB), lambda m, k: (k, 0, 0)),
            pl.BlockSpec((MB, 1), lambda m, k: (m, 0)),
        ],
        out_specs=pl.BlockSpec((n_out, MB, FB), lambda m, k: (0, m, 0)),
        out_shape=jax.ShapeDtypeStruct((n_out, N, FB), jnp.float32),
        scratch_shapes=[pltpu.VMEM((MB, fpad), jnp.float32)],
        compiler_params=pltpu.CompilerParams(
            dimension_semantics=("parallel", "arbitrary")),
    )(acc_in, w, b2d, dinv2d)


def _final(acc_in, dinv2d, b6):
    def body(acc_ref, dinv_ref, b_ref, out_ref):
        a = acc_ref[0] + acc_ref[1]
        out_ref[...] = a[:, 0:1] * dinv_ref[...] + b_ref[...]

    return pl.pallas_call(
        body,
        grid=(N // MB,),
        in_specs=[
            pl.BlockSpec((2, MB, FB), lambda m: (0, m, 0)),
            pl.BlockSpec((MB, 1), lambda m: (m, 0)),
            pl.BlockSpec((1, 1), lambda m: (0, 0)),
        ],
        out_specs=pl.BlockSpec((MB, 1), lambda m: (m, 0)),
        out_shape=jax.ShapeDtypeStruct((N, 1), jnp.float32),
    )(acc_in, dinv2d, b6)


def kernel(x, edge_index, batch, W1, b1, W2, b2, W3, b3, W4, b4, W5, b5,
           W6, b6):
    del batch
    src2d = edge_index[0].reshape(NCHUNK, 1, CH)
    dst2d = edge_index[1].reshape(NCHUNK, 1, CH)
    init2 = jnp.stack([jnp.ones((N,), jnp.float32),
                       jnp.zeros((N,), jnp.float32)])

    deg = _degree_kernel(dst2d, init2)
    nm2, dinv2d = _norm_dinv(x, deg.reshape(2, N, 1))

    Ws = [W1, W2, W3, W4, W5, W6]
    bs = [b1, b2, b3, b4, b5, b6]

    w = jnp.pad(W1, ((0, KPAD[0] - DIMS_K[0]), (0, FPAD[0] - DIMS_F[0])))
    hs = _mm_layer1(x, w, dinv2d, nm2)
    acc = _make_scatter(NFB[0])(hs, src2d, dst2d)

    for l in range(1, 6):
        w = jnp.pad(Ws[l], ((0, KPAD[l] - DIMS_K[l]), (0, FPAD[l] - DIMS_F[l])))
        b2d = jnp.pad(bs[l - 1], (0, KPAD[l] - DIMS_K[l])).reshape(NKB[l], 1, FB)
        hs = _mm_layer(l, acc, w, b2d, dinv2d)
        acc = _make_scatter(NFB[l])(hs, src2d, dst2d)

    out = _final(acc, dinv2d, b6.reshape(1, 1))
    return out.reshape(1, N)


# speedup vs baseline: 7.4477x; 1.0603x over previous
import functools

import jax
import jax.numpy as jnp
from jax import lax
from jax.experimental import pallas as pl
from jax.experimental.pallas import tpu as pltpu
from jax.experimental.pallas import tpu_sc as plsc

N = 10000
E = 160000
FEAT = 3244
MB = 1000
FB = 128
CH = 125
NCHUNK = E // CH
SC_TILES = 16
ROWS_PER_TILE = NCHUNK // SC_TILES

DIMS_K = [FEAT, 2000, 1000, 500, 100, 10]
DIMS_F = [2000, 1000, 500, 100, 10, 1]
KPAD = [3328, 2048, 1024, 512, 128, 128]
FPAD = [2048, 1024, 512, 128, 128, 128]
NKB = [kp // FB for kp in KPAD]
NFB = [fp // FB for fp in FPAD]

@functools.lru_cache(maxsize=None)
def _sc_mesh():
    return plsc.VectorSubcoreMesh(
        core_axis_name="c", subcore_axis_name="s",
        num_cores=2, num_subcores=16)


def _degree_kernel(dst2d, init2):
    @functools.partial(
        pl.kernel,
        out_type=jax.ShapeDtypeStruct((2, N), jnp.float32),
        mesh=_sc_mesh(),
        scratch_types=[
            pltpu.VMEM_SHARED((N,), jnp.float32),
            pltpu.VMEM((1, CH), jnp.int32),
            pltpu.VMEM((CH,), jnp.float32),
        ],
    )
    def deg_kernel(dst_hbm, init_hbm, deg_hbm, deg_sp, dst_v, ones_v):
        c = lax.axis_index("c")
        s = lax.axis_index("s")
        rows = ROWS_PER_TILE // 2
        row0 = (c * SC_TILES + s) * rows
        pltpu.sync_copy(init_hbm.at[0, pl.ds(0, CH)], ones_v)

        @pl.when(s == 0)
        def _():
            pltpu.sync_copy(init_hbm.at[c], deg_sp)

        plsc.subcore_barrier()

        def body(j, carry):
            pltpu.sync_copy(dst_hbm.at[row0 + j], dst_v)
            pltpu.sync_copy(ones_v, deg_sp.at[dst_v.at[0]], add=True)
            return carry

        lax.fori_loop(0, rows, body, 0)
        plsc.subcore_barrier()

        @pl.when(s == 0)
        def _():
            pltpu.sync_copy(deg_sp, deg_hbm.at[c])

    return deg_kernel(dst2d, init2)


def _edge_pipeline(blk, acc_sp, row0, nchunks, src_hbm, dst_hbm,
                   srcb, dstb, rows, isem, gsem, ssem):
    nit = nchunks // 6
    nfull = nit * 6

    def _idx(c, xs):
        pltpu.async_copy(src_hbm.at[c], srcb[xs], isem[xs])
        pltpu.async_copy(dst_hbm.at[c], dstb[xs], isem[xs])

    def _wait_idx(c, xs):
        pltpu.make_async_copy(src_hbm.at[c], srcb[xs], isem[xs]).wait()
        pltpu.make_async_copy(dst_hbm.at[c], dstb[xs], isem[xs]).wait()

    def _fire_gather(rs, xs):
        pltpu.async_copy(blk.at[srcb[xs].at[0]], rows[rs], gsem[rs])

    def _wait_gather(rs, xs):
        pltpu.make_async_copy(blk.at[srcb[xs].at[0]], rows[rs],
                              gsem[rs]).wait()

    def _fire_scatter(rs, xs):
        pltpu.async_copy(rows[rs], acc_sp.at[dstb[xs].at[0]], ssem[rs],
                         add=True)

    def _wait_scatter(rs, xs):
        pltpu.make_async_copy(rows[rs], acc_sp.at[dstb[xs].at[0]],
                              ssem[rs]).wait()

    for t in range(5):
        _idx(row0 + t, t)
    _wait_idx(row0, 0)
    _fire_gather(0, 0)
    _wait_idx(row0 + 1, 1)
    _fire_gather(1, 1)

    def body(i, carry):
        j = row0 + 6 * i
        for t in range(6):
            c = j + t
            rs, xs = t % 3, t
            rs2, xs2 = (t + 2) % 3, (t + 2) % 6
            xs_prev = (t + 5) % 6

            def _advance(c=c, rs2=rs2, xs2=xs2, xs_prev=xs_prev,
                         skip_wait=False):
                if not skip_wait:
                    _wait_scatter(rs2, xs_prev)
                _wait_idx(c + 2, xs2)
                _fire_gather(rs2, xs2)

            if t == 0:
                @pl.when(i > 0)
                def _():
                    _advance()

                @pl.when(i == 0)
                def _():
                    _advance(skip_wait=True)

                _idx(c + 5, xs_prev)
            elif t >= 4:
                @pl.when(i < nit - 1)
                def _():
                    _advance()
                    _idx(c + 5, xs_prev)
            else:
                _advance()

                @pl.when(i < nit - 1)
                def _():
                    _idx(c + 5, xs_prev)

            _wait_gather(rs, xs)
            _fire_scatter(rs, xs)
        return carry

    lax.fori_loop(0, nit, body, 0)

    for k in range(3):
        _wait_scatter(k, 3 + k)

    for u in range(nchunks - nfull):
        c = row0 + nfull + u
        _idx(c, u)
        _wait_idx(c, u)
        _fire_gather(u % 3, u)
        _wait_gather(u % 3, u)
        pltpu.sync_copy(rows[u % 3], acc_sp.at[dstb[u].at[0]], add=True)


@functools.lru_cache(maxsize=None)
def _make_scatter(n_fb):
    n_out = max(n_fb, 2)

    @functools.partial(
        pl.kernel,
        out_type=jax.ShapeDtypeStruct((n_out, N, FB), jnp.float32),
        mesh=_sc_mesh(),
        scratch_types=(
            [pltpu.VMEM_SHARED((N, FB), jnp.float32)]
            + [pltpu.VMEM((1, CH), jnp.int32)] * 12
            + [pltpu.VMEM((CH, FB), jnp.float32)] * 3
            + [pltpu.SemaphoreType.DMA] * 12
        ),
    )
    def scat_kernel(hs_hbm, src_hbm, dst_hbm, acc_hbm, acc_sp, *sc):
        c = lax.axis_index("c")
        s = lax.axis_index("s")
        bufs = (list(sc[0:6]), list(sc[6:12]), list(sc[12:15]),
                list(sc[15:21]), list(sc[21:24]), list(sc[24:27]))
        if n_fb == 1:
            rows = ROWS_PER_TILE // 2
            row0 = (c * SC_TILES + s) * rows

            @pl.when(s == 0)
            def _():
                pltpu.sync_copy(hs_hbm.at[c], acc_sp)

            plsc.subcore_barrier()
            _edge_pipeline(hs_hbm.at[0], acc_sp, row0, rows,
                           src_hbm, dst_hbm, *bufs)
            plsc.subcore_barrier()

            @pl.when(s == 0)
            def _():
                pltpu.sync_copy(acc_sp, acc_hbm.at[c])
        else:
            row0 = s * ROWS_PER_TILE
            for fb in range(n_fb):
                owner = fb % 2

                @pl.when(c == owner)
                def _(fb=fb):
                    @pl.when(s == 0)
                    def _():
                        pltpu.sync_copy(hs_hbm.at[fb], acc_sp)

                    plsc.subcore_barrier()
                    _edge_pipeline(hs_hbm.at[fb], acc_sp, row0,
                                   ROWS_PER_TILE,
                                   src_hbm, dst_hbm, *bufs)
                    plsc.subcore_barrier()

                    @pl.when(s == 0)
                    def _():
                        pltpu.sync_copy(acc_sp, acc_hbm.at[fb])

                    plsc.subcore_barrier()

    return scat_kernel


def _norm_dinv(x, deg3d):
    def body(x_ref, deg_ref, nm2_ref, dinv_ref, acc_ref):
        i = pl.program_id(0)

        @pl.when(i == 0)
        def _():
            acc_ref[...] = jnp.zeros_like(acc_ref)

        xb = x_ref[...]
        acc_ref[...] += jnp.sum(xb * xb).reshape(1, 1)
        dinv_ref[...] = lax.rsqrt(deg_ref[0] + deg_ref[1])

        @pl.when(i == pl.num_programs(0) - 1)
        def _():
            nm2_ref[...] = acc_ref[...]

    return pl.pallas_call(
        body,
        grid=(N // MB,),
        in_specs=[
            pl.BlockSpec((MB, FEAT), lambda i: (i, 0)),
            pl.BlockSpec((2, MB, 1), lambda i: (0, i, 0)),
        ],
        out_specs=[
            pl.BlockSpec((1, 1), lambda i: (0, 0)),
            pl.BlockSpec((MB, 1), lambda i: (i, 0)),
        ],
        out_shape=[
            jax.ShapeDtypeStruct((1, 1), jnp.float32),
            jax.ShapeDtypeStruct((N, 1), jnp.float32),
        ],
        scratch_shapes=[pltpu.VMEM((1, 1), jnp.float32)],
        compiler_params=pltpu.CompilerParams(
            dimension_semantics=("arbitrary",)),
    )(x, deg3d)


def _mm_layer1(x, w, dinv2d, nm2):
    nfb = NFB[0]
    fpad = FPAD[0]
    kb1 = 256
    nkb = KPAD[0] // kb1

    def body(x_ref, w_ref, dinv_ref, nm2_ref, out_ref, acc_ref):
        k = pl.program_id(1)

        @pl.when(k == 0)
        def _():
            acc_ref[...] = jnp.zeros_like(acc_ref)

        xb = x_ref[...]
        col = k * kb1 + lax.broadcasted_iota(jnp.int32, xb.shape, 1)
        xb = jnp.where(col < FEAT, xb, 0.0)
        acc_ref[...] += jnp.dot(xb, w_ref[...],
                                preferred_element_type=jnp.float32)

        @pl.when(k == nkb - 1)
        def _():
            scale = dinv_ref[...] * lax.rsqrt(nm2_ref[...])
            for f in range(nfb):
                out_ref[f] = acc_ref[:, f * FB:(f + 1) * FB] * scale

    return pl.pallas_call(
        body,
        grid=(N // MB, nkb),
        in_specs=[
            pl.BlockSpec((MB, kb1), lambda m, k: (m, k)),
            pl.BlockSpec((kb1, fpad), lambda m, k: (k, 0)),
            pl.BlockSpec((MB, 1), lambda m, k: (m, 0)),
            pl.BlockSpec((1, 1), lambda m, k: (0, 0)),
        ],
        out_specs=pl.BlockSpec((nfb, MB, FB), lambda m, k: (0, m, 0)),
        out_shape=jax.ShapeDtypeStruct((nfb, N, FB), jnp.float32),
        scratch_shapes=[pltpu.VMEM((MB, fpad), jnp.float32)],
        compiler_params=pltpu.CompilerParams(
            dimension_semantics=("parallel", "arbitrary")),
    )(x, w, dinv2d, nm2)


def _mm_layer(layer, acc_in, w, b2d, dinv2d):
    nkb, nfb = NKB[layer], NFB[layer]
    fpad = FPAD[layer]
    parts_in = NFB[layer - 1] == 1
    parts_out = nfb == 1

    def body(x_ref, w_ref, b_ref, dinv_ref, out_ref, acc_ref):
        k = pl.program_id(1)

        @pl.when(k == 0)
        def _():
            acc_ref[...] = jnp.zeros_like(acc_ref)

        dv = dinv_ref[...]
        xin = x_ref[0] + x_ref[1] if parts_in else x_ref[0]
        xb = jnp.maximum(xin * dv + b_ref[0], 0.0)
        acc_ref[...] += jnp.dot(xb, w_ref[...],
                                preferred_element_type=jnp.float32)

        @pl.when(k == nkb - 1)
        def _():
            if parts_out:
                out_ref[0] = acc_ref[...] * dv
                out_ref[1] = jnp.zeros_like(acc_ref)
            else:
                for f in range(nfb):
                    out_ref[f] = acc_ref[:, f * FB:(f + 1) * FB] * dv

    in_blk = (2, MB, FB) if parts_in else (1, MB, FB)
    in_map = (lambda m, k: (0, m, 0)) if parts_in else (lambda m, k: (k, m, 0))
    n_out = max(nfb, 2) if parts_out else nfb
    return pl.pallas_call(
        body,
        grid=(N // MB, nkb),
        in_specs=[
            pl.BlockSpec(in_blk, in_map),
            pl.BlockSpec((FB, fpad), lambda m, k: (k, 0)),
            pl.BlockSpec((1, 1, FB), lambda m, k: (k, 0, 0)),
            pl.BlockSpec((MB, 1), lambda m, k: (m, 0)),
        ],
        out_specs=pl.BlockSpec((n_out, MB, FB), lambda m, k: (0, m, 0)),
        out_shape=jax.ShapeDtypeStruct((n_out, N, FB), jnp.float32),
        scratch_shapes=[pltpu.VMEM((MB, fpad), jnp.float32)],
        compiler_params=pltpu.CompilerParams(
            dimension_semantics=("parallel", "arbitrary")),
    )(acc_in, w, b2d, dinv2d)


def _final(acc_in, dinv2d, b6):
    def body(acc_ref, dinv_ref, b_ref, out_ref):
        a = acc_ref[0] + acc_ref[1]
        out_ref[...] = a[:, 0:1] * dinv_ref[...] + b_ref[...]

    return pl.pallas_call(
        body,
        grid=(N // MB,),
        in_specs=[
            pl.BlockSpec((2, MB, FB), lambda m: (0, m, 0)),
            pl.BlockSpec((MB, 1), lambda m: (m, 0)),
            pl.BlockSpec((1, 1), lambda m: (0, 0)),
        ],
        out_specs=pl.BlockSpec((MB, 1), lambda m: (m, 0)),
        out_shape=jax.ShapeDtypeStruct((N, 1), jnp.float32),
    )(acc_in, dinv2d, b6)


def kernel(x, edge_index, batch, W1, b1, W2, b2, W3, b3, W4, b4, W5, b5,
           W6, b6):
    del batch
    src2d = edge_index[0].reshape(NCHUNK, 1, CH)
    dst2d = edge_index[1].reshape(NCHUNK, 1, CH)
    init2 = jnp.stack([jnp.ones((N,), jnp.float32),
                       jnp.zeros((N,), jnp.float32)])

    deg = _degree_kernel(dst2d, init2)
    nm2, dinv2d = _norm_dinv(x, deg.reshape(2, N, 1))

    Ws = [W1, W2, W3, W4, W5, W6]
    bs = [b1, b2, b3, b4, b5, b6]

    w = jnp.pad(W1, ((0, KPAD[0] - DIMS_K[0]), (0, FPAD[0] - DIMS_F[0])))
    hs = _mm_layer1(x, w, dinv2d, nm2)
    acc = _make_scatter(NFB[0])(hs, src2d, dst2d)

    for l in range(1, 6):
        w = jnp.pad(Ws[l], ((0, KPAD[l] - DIMS_K[l]), (0, FPAD[l] - DIMS_F[l])))
        b2d = jnp.pad(bs[l - 1], (0, KPAD[l] - DIMS_K[l])).reshape(NKB[l], 1, FB)
        hs = _mm_layer(l, acc, w, b2d, dinv2d)
        acc = _make_scatter(NFB[l])(hs, src2d, dst2d)

    out = _final(acc, dinv2d, b6.reshape(1, 1))
    return out.reshape(1, N)
